# baseline (speedup 1.0000x reference)
import functools

import numpy as np
import jax
import jax.numpy as jnp
from jax import lax
from jax.experimental import pallas as pl
from jax.experimental.pallas import tpu as pltpu
from jax.experimental.pallas import tpu_sc as plsc

_ALPHA = 0.5
_THETA = 1.0

_NC = 2
_NS = 16
_NW = _NC * _NS
_CHUNK = 128
_SUB = 8


def _elu(z):
    return jnp.where(z > 0.0, z, jnp.exp(jnp.minimum(z, 0.0)) - 1.0)


def _acc_rows(n):
    per_tile = ((n + 1 + _NS * _CHUNK - 1) // (_NS * _CHUNK)) * _CHUNK
    return _NS * per_tile


def _deg_count_sc(dst2d, n_acc, ngrp, h):
    mesh = plsc.VectorSubcoreMesh(core_axis_name="c", subcore_axis_name="s")
    zpt = n_acc // _NS

    scratch = (
        [pltpu.VMEM((_SUB, _CHUNK), jnp.int32) for _ in range(2)]
        + [pltpu.VMEM((_CHUNK, h), jnp.float32),
           pltpu.VMEM((_CHUNK, h), jnp.float32),
           pltpu.VMEM_SHARED((n_acc, h), jnp.float32)]
        + [pltpu.SemaphoreType.DMA for _ in range(3)]
    )

    @functools.partial(
        pl.kernel,
        out_type=jax.ShapeDtypeStruct((_NC, n_acc, h), jnp.float32),
        mesh=mesh,
        scratch_types=scratch,
        compiler_params=pltpu.CompilerParams(use_tc_tiling_on_sc=False),
    )
    def deg_kernel(dst_hbm, out_hbm, dst_a, dst_b, ones_v, zrows, acc,
                   ssem, isem_a, isem_b):
        c = lax.axis_index("c")
        s = lax.axis_index("s")
        w = s * _NC + c

        zero16 = jnp.zeros((16,), jnp.float32)
        ones16 = jnp.ones((16,), jnp.float32)

        @pl.loop(0, _CHUNK)
        def _(i):
            zrows[i, :] = zero16
            ones_v[i, :] = ones16

        @pl.loop(0, zpt // _CHUNK)
        def _(i):
            pltpu.sync_copy(zrows, acc.at[pl.ds(s * zpt + i * _CHUNK, _CHUNK)])

        plsc.subcore_barrier()

        cbase = w * ngrp * _SUB

        def fire_idx(gidx, dbuf, sem):
            pltpu.async_copy(dst_hbm.at[pl.ds(cbase + gidx * _SUB, _SUB)],
                             dbuf, sem)

        def wait_idx(dbuf, sem):
            pltpu.make_async_copy(dst_hbm.at[pl.ds(0, _SUB)], dbuf, sem).wait()

        def drain_scatters():
            for j in range(_SUB):
                pltpu.make_async_copy(
                    dst_hbm.at[pl.ds(0, _SUB)], ones_v, ssem).wait()

        def run_group(dbuf):
            for j in range(_SUB):
                pltpu.async_copy(ones_v, acc.at[dbuf.at[j]], ssem, add=True)

        fire_idx(0, dst_a, isem_a)

        @pl.loop(0, ngrp // 2)
        def _(i):
            g_a = 2 * i
            wait_idx(dst_a, isem_a)

            @pl.when(i > 0)
            def _():
                drain_scatters()

            fire_idx(g_a + 1, dst_b, isem_b)
            run_group(dst_a)
            wait_idx(dst_b, isem_b)
            drain_scatters()
            fire_idx(jnp.minimum(g_a + 2, ngrp - 1), dst_a, isem_a)
            run_group(dst_b)

        drain_scatters()
        wait_idx(dst_a, isem_a)
        plsc.subcore_barrier()
        pltpu.sync_copy(acc.at[pl.ds(s * zpt, zpt)],
                        out_hbm.at[c, pl.ds(s * zpt, zpt)])

    return deg_kernel(dst2d)


def _seg_sum_sc(g, src2d, dst2d, n_acc, ngrp, h):
    mesh = plsc.VectorSubcoreMesh(core_axis_name="c", subcore_axis_name="s")
    zpt = n_acc // _NS

    scratch = (
        [pltpu.VMEM((_SUB, _CHUNK), jnp.int32) for _ in range(4)]
        + [pltpu.VMEM((_CHUNK, h), jnp.float32) for _ in range(_SUB)]
        + [pltpu.VMEM((_CHUNK, h), jnp.float32),
           pltpu.VMEM_SHARED((n_acc, h), jnp.float32)]
        + [pltpu.SemaphoreType.DMA for _ in range(_SUB + 3)]
    )

    @functools.partial(
        pl.kernel,
        out_type=jax.ShapeDtypeStruct((_NC, n_acc, h), jnp.float32),
        mesh=mesh,
        scratch_types=scratch,
        compiler_params=pltpu.CompilerParams(use_tc_tiling_on_sc=False),
    )
    def seg_kernel(g_hbm, src_hbm, dst_hbm, out_hbm, *scr):
        src_a, dst_a, src_b, dst_b = scr[0:4]
        bufs = scr[4:4 + _SUB]
        zrows = scr[4 + _SUB]
        acc = scr[5 + _SUB]
        gsems = scr[6 + _SUB:6 + 2 * _SUB]
        ssem = scr[6 + 2 * _SUB]
        isem_a = scr[7 + 2 * _SUB]
        isem_b = scr[8 + 2 * _SUB]

        c = lax.axis_index("c")
        s = lax.axis_index("s")
        w = s * _NC + c

        zero16 = jnp.zeros((16,), jnp.float32)

        @pl.loop(0, _CHUNK)
        def _(i):
            zrows[i, :] = zero16

        @pl.loop(0, zpt // _CHUNK)
        def _(i):
            pltpu.sync_copy(zrows, acc.at[pl.ds(s * zpt + i * _CHUNK, _CHUNK)])

        plsc.subcore_barrier()

        cbase = w * ngrp * _SUB

        def fire_idx(gidx, sbuf, dbuf, sem):
            rows = pl.ds(cbase + gidx * _SUB, _SUB)
            pltpu.async_copy(src_hbm.at[rows], sbuf, sem)
            pltpu.async_copy(dst_hbm.at[rows], dbuf, sem)

        def wait_idx(sbuf, dbuf, sem):
            pltpu.make_async_copy(src_hbm.at[pl.ds(0, _SUB)], sbuf, sem).wait()
            pltpu.make_async_copy(src_hbm.at[pl.ds(0, _SUB)], dbuf, sem).wait()

        def drain_scatters():
            for j in range(_SUB):
                pltpu.make_async_copy(
                    g_hbm.at[pl.ds(0, _CHUNK)], bufs[j], ssem).wait()

        def run_group(sbuf, dbuf):
            descs = [pltpu.async_copy(g_hbm.at[sbuf.at[j]], bufs[j], gsems[j])
                     for j in range(_SUB)]
            for j in range(_SUB):
                descs[j].wait()
                pltpu.async_copy(bufs[j], acc.at[dbuf.at[j]], ssem, add=True)

        fire_idx(0, src_a, dst_a, isem_a)

        @pl.loop(0, ngrp // 2)
        def _(i):
            g_a = 2 * i
            wait_idx(src_a, dst_a, isem_a)

            @pl.when(i > 0)
            def _():
                drain_scatters()

            fire_idx(g_a + 1, src_b, dst_b, isem_b)
            run_group(src_a, dst_a)
            wait_idx(src_b, dst_b, isem_b)
            drain_scatters()
            fire_idx(jnp.minimum(g_a + 2, ngrp - 1), src_a, dst_a, isem_a)
            run_group(src_b, dst_b)

        drain_scatters()
        wait_idx(src_a, dst_a, isem_a)
        plsc.subcore_barrier()
        pltpu.sync_copy(acc.at[pl.ds(s * zpt, zpt)],
                        out_hbm.at[c, pl.ds(s * zpt, zpt)])

    return seg_kernel(g, src2d, dst2d)


def _init_tc(x3, w_cols, b128, deglin, bn8):
    nr = x3.shape[0]

    def body(*refs):
        x_b = refs[0]
        wks = refs[1:9]
        b_b, dg_b = refs[9], refs[10]
        h_b, g_b, di_b = refs[11], refs[12], refs[13]
        dinv = lax.rsqrt(dg_b[0] + dg_b[1] + 1.0)
        acc = jnp.broadcast_to(b_b[:], (x_b.shape[0], 128))
        for k in range(8):
            acc = acc + jnp.dot(x_b[:, k, :], wks[k][:],
                                preferred_element_type=jnp.float32)
        v = _elu(acc)
        h_b[:] = v
        g_b[:] = dinv * v
        di_b[:] = dinv

    w_specs = [pl.BlockSpec((128, 128), lambda i: (0, 0)) for _ in range(8)]
    row = lambda i: (i, 0)
    return pl.pallas_call(
        body,
        grid=(nr // bn8,),
        in_specs=[pl.BlockSpec((bn8, 8, 128), lambda i: (i, 0, 0))]
        + w_specs + [
            pl.BlockSpec((1, 128), lambda i: (0, 0)),
            pl.BlockSpec((2, bn8, 128), lambda i: (0, i, 0)),
        ],
        out_specs=[pl.BlockSpec((bn8, 128), row)] * 3,
        out_shape=[jax.ShapeDtypeStruct((nr, 128), jnp.float32)] * 3,
    )(x3, *w_cols, b128, deglin)


def _layer_tc(partlin, hlin, h0lin, dinvlin, w128, beta, bn8):
    nr = hlin.shape[0]

    def body(p_b, h_b, h0_b, di_b, w_b, hn_b, gn_b):
        dinv = di_b[:]
        ah = dinv * (p_b[0] + p_b[1]) + (dinv * dinv) * h_b[:]
        hh = (1.0 - _ALPHA) * ah + _ALPHA * h0_b[:]
        out = (1.0 - beta) * hh + beta * jnp.dot(
            hh, w_b[:], preferred_element_type=jnp.float32)
        hn = _elu(out) + out
        hn_b[:] = hn
        gn_b[:] = dinv * hn

    row = lambda i: (i, 0)
    return pl.pallas_call(
        body,
        grid=(nr // bn8,),
        in_specs=[
            pl.BlockSpec((2, bn8, 128), lambda i: (0, i, 0)),
            pl.BlockSpec((bn8, 128), row),
            pl.BlockSpec((bn8, 128), row),
            pl.BlockSpec((bn8, 128), row),
            pl.BlockSpec((128, 128), lambda i: (0, 0)),
        ],
        out_specs=[pl.BlockSpec((bn8, 128), row), pl.BlockSpec((bn8, 128), row)],
        out_shape=[jax.ShapeDtypeStruct((nr, 128), jnp.float32)] * 2,
    )(partlin, hlin, h0lin, dinvlin, w128)


def _final_tc(partlin, hlin, h0lin, dinvlin, w128, wo512, bo512, beta, bn8):
    nr = hlin.shape[0]

    def body(p_b, h_b, h0_b, di_b, w_b, wo_b, bo_b, y_b):
        dinv = di_b[:]
        ah = dinv * (p_b[0] + p_b[1]) + (dinv * dinv) * h_b[:]
        hh = (1.0 - _ALPHA) * ah + _ALPHA * h0_b[:]
        out = (1.0 - beta) * hh + beta * jnp.dot(
            hh, w_b[:], preferred_element_type=jnp.float32)
        hn = _elu(out) + out
        y_b[:] = jnp.dot(hn, wo_b[:],
                         preferred_element_type=jnp.float32) + bo_b[:]

    row = lambda i: (i, 0)
    return pl.pallas_call(
        body,
        grid=(nr // bn8,),
        in_specs=[
            pl.BlockSpec((2, bn8, 128), lambda i: (0, i, 0)),
            pl.BlockSpec((bn8, 128), row),
            pl.BlockSpec((bn8, 128), row),
            pl.BlockSpec((bn8, 128), row),
            pl.BlockSpec((128, 128), lambda i: (0, 0)),
            pl.BlockSpec((128, 512), lambda i: (0, 0)),
            pl.BlockSpec((1, 512), lambda i: (0, 0)),
        ],
        out_specs=pl.BlockSpec((bn8, 512), row),
        out_shape=jax.ShapeDtypeStruct((nr, 512), jnp.float32),
    )(partlin, hlin, h0lin, dinvlin, w128, wo512, bo512)


def _block_diag8(w):
    a, b = w.shape
    out = jnp.zeros((8 * a, 8 * b), w.dtype)
    for k in range(8):
        out = out.at[k * a:(k + 1) * a, k * b:(k + 1) * b].set(w)
    return out


def kernel(x, edge_index, W_in, b_in, W_layers, W_out, b_out):
    n, fin = x.shape
    e = edge_index.shape[1]
    h = W_in.shape[1]
    n_layers = W_layers.shape[0]
    co = W_out.shape[1]

    grand = _NW * _SUB * _CHUNK
    ngrp = (e + grand - 1) // grand
    ngrp += ngrp % 2
    e_pad = ngrp * grand
    n_acc = _acc_rows(n)
    nr = n_acc * h // 128

    src = edge_index[0]
    dst = edge_index[1]
    pad = e_pad - e
    src2d = jnp.concatenate(
        [src, jnp.zeros((pad,), jnp.int32)]).reshape(-1, _CHUNK)
    dst2d = jnp.concatenate(
        [dst, jnp.full((pad,), n, jnp.int32)]).reshape(-1, _CHUNK)

    degp = _deg_count_sc(dst2d, n_acc, ngrp, h)
    deglin = degp.reshape(2, nr, 128)

    x3 = jnp.pad(x, ((0, n_acc - n), (0, 0))).reshape(nr, 8, fin)
    w_cols = [jnp.zeros((fin, 128), jnp.float32)
              .at[:, 16 * k:16 * k + h].set(W_in) for k in range(8)]
    b128 = jnp.tile(b_in, 8).reshape(1, 128)
    wo512 = _block_diag8(W_out)
    bo512 = jnp.tile(b_out, 8).reshape(1, 8 * co)

    bn8 = 1568
    hlin, glin, dinvlin = _init_tc(x3, w_cols, b128, deglin, bn8)
    h0lin = hlin
    y = None
    for i in range(n_layers):
        part = _seg_sum_sc(glin.reshape(n_acc, h), src2d, dst2d,
                           n_acc, ngrp, h)
        partlin = part.reshape(2, nr, 128)
        beta = float(np.log(_THETA / (i + 1) + 1.0))
        w128 = _block_diag8(W_layers[i])
        if i + 1 < n_layers:
            hlin, glin = _layer_tc(partlin, hlin, h0lin, dinvlin,
                                   w128, beta, bn8)
        else:
            y = _final_tc(partlin, hlin, h0lin, dinvlin, w128,
                          wo512, bo512, beta, bn8)
    return y.reshape(n_acc, co)[:n]


# speedup vs baseline: 1.1779x; 1.1779x over previous
import functools

import numpy as np
import jax
import jax.numpy as jnp
from jax import lax
from jax.experimental import pallas as pl
from jax.experimental.pallas import tpu as pltpu
from jax.experimental.pallas import tpu_sc as plsc

_ALPHA = 0.5
_THETA = 1.0

_NC = 2
_NS = 16
_NW = _NC * _NS
_CHUNK = 128
_SUB = 8


def _elu(z):
    return jnp.where(z > 0.0, z, jnp.exp(jnp.minimum(z, 0.0)) - 1.0)


def _acc_rows(n):
    per_tile = ((n + 1 + _NS * _CHUNK - 1) // (_NS * _CHUNK)) * _CHUNK
    return _NS * per_tile


def _deg_count_sc(dst2d, n_acc, ngrp, h):
    mesh = plsc.VectorSubcoreMesh(core_axis_name="c", subcore_axis_name="s")
    zpt = n_acc // _NS

    scratch = (
        [pltpu.VMEM((_SUB, _CHUNK), jnp.int32) for _ in range(2)]
        + [pltpu.VMEM((_CHUNK, h), jnp.float32),
           pltpu.VMEM((_CHUNK, h), jnp.float32),
           pltpu.VMEM_SHARED((n_acc, h), jnp.float32)]
        + [pltpu.SemaphoreType.DMA for _ in range(3)]
    )

    @functools.partial(
        pl.kernel,
        out_type=jax.ShapeDtypeStruct((_NC, n_acc, h), jnp.float32),
        mesh=mesh,
        scratch_types=scratch,
        compiler_params=pltpu.CompilerParams(use_tc_tiling_on_sc=False),
    )
    def deg_kernel(dst_hbm, out_hbm, dst_a, dst_b, ones_v, zrows, acc,
                   ssem, isem_a, isem_b):
        c = lax.axis_index("c")
        s = lax.axis_index("s")
        w = s * _NC + c

        zero16 = jnp.zeros((16,), jnp.float32)
        ones16 = jnp.ones((16,), jnp.float32)

        @pl.loop(0, _CHUNK)
        def _(i):
            zrows[i, :] = zero16
            ones_v[i, :] = ones16

        @pl.loop(0, zpt // _CHUNK)
        def _(i):
            pltpu.sync_copy(zrows, acc.at[pl.ds(s * zpt + i * _CHUNK, _CHUNK)])

        plsc.subcore_barrier()

        cbase = w * ngrp * _SUB

        def fire_idx(gidx, dbuf, sem):
            pltpu.async_copy(dst_hbm.at[pl.ds(cbase + gidx * _SUB, _SUB)],
                             dbuf, sem)

        def wait_idx(dbuf, sem):
            pltpu.make_async_copy(dst_hbm.at[pl.ds(0, _SUB)], dbuf, sem).wait()

        def drain_scatters():
            for j in range(_SUB):
                pltpu.make_async_copy(
                    dst_hbm.at[pl.ds(0, _SUB)], ones_v, ssem).wait()

        def run_group(dbuf):
            for j in range(_SUB):
                pltpu.async_copy(ones_v, acc.at[dbuf.at[j]], ssem, add=True)

        fire_idx(0, dst_a, isem_a)

        @pl.loop(0, ngrp // 2)
        def _(i):
            g_a = 2 * i
            wait_idx(dst_a, isem_a)

            @pl.when(i > 0)
            def _():
                drain_scatters()

            fire_idx(g_a + 1, dst_b, isem_b)
            run_group(dst_a)
            wait_idx(dst_b, isem_b)
            drain_scatters()
            fire_idx(jnp.minimum(g_a + 2, ngrp - 1), dst_a, isem_a)
            run_group(dst_b)

        drain_scatters()
        wait_idx(dst_a, isem_a)
        plsc.subcore_barrier()
        pltpu.sync_copy(acc.at[pl.ds(s * zpt, zpt)],
                        out_hbm.at[c, pl.ds(s * zpt, zpt)])

    return deg_kernel(dst2d)


def _seg_sum_sc(g, src2d, dst2d, n_acc, ngrp, h):
    mesh = plsc.VectorSubcoreMesh(core_axis_name="c", subcore_axis_name="s")
    zpt = n_acc // _NS

    scratch = (
        [pltpu.VMEM((_SUB, _CHUNK), jnp.int32) for _ in range(4)]
        + [pltpu.VMEM((_CHUNK, h), jnp.float32) for _ in range(_SUB)]
        + [pltpu.VMEM((_CHUNK, h), jnp.float32),
           pltpu.VMEM_SHARED((n_acc, h), jnp.float32)]
        + [pltpu.SemaphoreType.DMA for _ in range(_SUB + 3)]
    )

    @functools.partial(
        pl.kernel,
        out_type=jax.ShapeDtypeStruct((_NC, n_acc, h), jnp.float32),
        mesh=mesh,
        scratch_types=scratch,
        compiler_params=pltpu.CompilerParams(use_tc_tiling_on_sc=False),
    )
    def seg_kernel(g_hbm, src_hbm, dst_hbm, out_hbm, *scr):
        src_a, dst_a, src_b, dst_b = scr[0:4]
        bufs = scr[4:4 + _SUB]
        zrows = scr[4 + _SUB]
        acc = scr[5 + _SUB]
        gsems = scr[6 + _SUB:6 + 2 * _SUB]
        ssem = scr[6 + 2 * _SUB]
        isem_a = scr[7 + 2 * _SUB]
        isem_b = scr[8 + 2 * _SUB]

        c = lax.axis_index("c")
        s = lax.axis_index("s")
        w = s * _NC + c

        zero16 = jnp.zeros((16,), jnp.float32)

        @pl.loop(0, _CHUNK)
        def _(i):
            zrows[i, :] = zero16

        @pl.loop(0, zpt // _CHUNK)
        def _(i):
            pltpu.sync_copy(zrows, acc.at[pl.ds(s * zpt + i * _CHUNK, _CHUNK)])

        plsc.subcore_barrier()

        cbase = w * ngrp * _SUB

        def fire_idx(gidx, sbuf, dbuf, sem):
            rows = pl.ds(cbase + gidx * _SUB, _SUB)
            pltpu.async_copy(src_hbm.at[rows], sbuf, sem)
            pltpu.async_copy(dst_hbm.at[rows], dbuf, sem)

        def wait_idx(sbuf, dbuf, sem):
            pltpu.make_async_copy(src_hbm.at[pl.ds(0, _SUB)], sbuf, sem).wait()
            pltpu.make_async_copy(src_hbm.at[pl.ds(0, _SUB)], dbuf, sem).wait()

        def drain_scatters():
            for j in range(_SUB):
                pltpu.make_async_copy(
                    g_hbm.at[pl.ds(0, _CHUNK)], bufs[j], ssem).wait()

        def run_group(sbuf, dbuf):
            descs = [pltpu.async_copy(g_hbm.at[sbuf.at[j]], bufs[j], gsems[j])
                     for j in range(_SUB)]
            for j in range(_SUB):
                descs[j].wait()
                pltpu.async_copy(bufs[j], acc.at[dbuf.at[j]], ssem, add=True)

        fire_idx(0, src_a, dst_a, isem_a)

        @pl.loop(0, ngrp // 2)
        def _(i):
            g_a = 2 * i
            wait_idx(src_a, dst_a, isem_a)

            @pl.when(i > 0)
            def _():
                drain_scatters()

            fire_idx(g_a + 1, src_b, dst_b, isem_b)
            run_group(src_a, dst_a)
            wait_idx(src_b, dst_b, isem_b)
            drain_scatters()
            fire_idx(jnp.minimum(g_a + 2, ngrp - 1), src_a, dst_a, isem_a)
            run_group(src_b, dst_b)

        drain_scatters()
        wait_idx(src_a, dst_a, isem_a)
        plsc.subcore_barrier()
        pltpu.sync_copy(acc.at[pl.ds(s * zpt, zpt)],
                        out_hbm.at[c, pl.ds(s * zpt, zpt)])

    return seg_kernel(g, src2d, dst2d)


def _init_tc(x3, w_cols, b128, deglin, bn8):
    nr = x3.shape[0]

    def body(*refs):
        x_b = refs[0]
        wks = refs[1:9]
        b_b, dg_b = refs[9], refs[10]
        h_b, g_b, di_b = refs[11], refs[12], refs[13]
        dinv = lax.rsqrt(dg_b[0] + dg_b[1] + 1.0)
        acc = jnp.broadcast_to(b_b[:], (x_b.shape[0], 128))
        for k in range(8):
            acc = acc + jnp.dot(x_b[:, k, :], wks[k][:],
                                preferred_element_type=jnp.float32)
        v = _elu(acc)
        h_b[:] = v
        g_b[:] = dinv * v
        di_b[:] = dinv

    w_specs = [pl.BlockSpec((128, 128), lambda i: (0, 0)) for _ in range(8)]
    row = lambda i: (i, 0)
    return pl.pallas_call(
        body,
        grid=(nr // bn8,),
        in_specs=[pl.BlockSpec((bn8, 8, 128), lambda i: (i, 0, 0))]
        + w_specs + [
            pl.BlockSpec((1, 128), lambda i: (0, 0)),
            pl.BlockSpec((2, bn8, 128), lambda i: (0, i, 0)),
        ],
        out_specs=[pl.BlockSpec((bn8, 128), row)] * 3,
        out_shape=[jax.ShapeDtypeStruct((nr, 128), jnp.float32)] * 3,
    )(x3, *w_cols, b128, deglin)


def _layer_tc(partlin, hlin, h0lin, dinvlin, w128, beta, bn8):
    nr = hlin.shape[0]

    def body(p_b, h_b, h0_b, di_b, w_b, hn_b, gn_b):
        dinv = di_b[:]
        ah = dinv * (p_b[0] + p_b[1]) + (dinv * dinv) * h_b[:]
        hh = (1.0 - _ALPHA) * ah + _ALPHA * h0_b[:]
        out = (1.0 - beta) * hh + beta * jnp.dot(
            hh, w_b[:], preferred_element_type=jnp.float32)
        hn = _elu(out) + out
        hn_b[:] = hn
        gn_b[:] = dinv * hn

    row = lambda i: (i, 0)
    return pl.pallas_call(
        body,
        grid=(nr // bn8,),
        in_specs=[
            pl.BlockSpec((2, bn8, 128), lambda i: (0, i, 0)),
            pl.BlockSpec((bn8, 128), row),
            pl.BlockSpec((bn8, 128), row),
            pl.BlockSpec((bn8, 128), row),
            pl.BlockSpec((128, 128), lambda i: (0, 0)),
        ],
        out_specs=[pl.BlockSpec((bn8, 128), row), pl.BlockSpec((bn8, 128), row)],
        out_shape=[jax.ShapeDtypeStruct((nr, 128), jnp.float32)] * 2,
    )(partlin, hlin, h0lin, dinvlin, w128)


def _final_tc(partlin, hlin, h0lin, dinvlin, w128, wo512, bo512, beta, bn8):
    nr = hlin.shape[0]

    def body(p_b, h_b, h0_b, di_b, w_b, wo_b, bo_b, y_b):
        dinv = di_b[:]
        ah = dinv * (p_b[0] + p_b[1]) + (dinv * dinv) * h_b[:]
        hh = (1.0 - _ALPHA) * ah + _ALPHA * h0_b[:]
        out = (1.0 - beta) * hh + beta * jnp.dot(
            hh, w_b[:], preferred_element_type=jnp.float32)
        hn = _elu(out) + out
        y_b[:] = jnp.dot(hn, wo_b[:],
                         preferred_element_type=jnp.float32) + bo_b[:]

    row = lambda i: (i, 0)
    return pl.pallas_call(
        body,
        grid=(nr // bn8,),
        in_specs=[
            pl.BlockSpec((2, bn8, 128), lambda i: (0, i, 0)),
            pl.BlockSpec((bn8, 128), row),
            pl.BlockSpec((bn8, 128), row),
            pl.BlockSpec((bn8, 128), row),
            pl.BlockSpec((128, 128), lambda i: (0, 0)),
            pl.BlockSpec((128, 512), lambda i: (0, 0)),
            pl.BlockSpec((1, 512), lambda i: (0, 0)),
        ],
        out_specs=pl.BlockSpec((bn8, 512), row),
        out_shape=jax.ShapeDtypeStruct((nr, 512), jnp.float32),
    )(partlin, hlin, h0lin, dinvlin, w128, wo512, bo512)


def _block_diag8(w):
    a, b = w.shape
    out = jnp.zeros((8 * a, 8 * b), w.dtype)
    for k in range(8):
        out = out.at[k * a:(k + 1) * a, k * b:(k + 1) * b].set(w)
    return out


def kernel(x, edge_index, W_in, b_in, W_layers, W_out, b_out):
    n, fin = x.shape
    e = edge_index.shape[1]
    h = W_in.shape[1]
    n_layers = W_layers.shape[0]
    co = W_out.shape[1]

    grand = _NW * _SUB * _CHUNK
    ngrp = (e + grand - 1) // grand
    ngrp += ngrp % 2
    e_pad = ngrp * grand
    n_acc = _acc_rows(n)
    nr = n_acc * h // 128

    src = edge_index[0]
    dst = edge_index[1]
    pad = e_pad - e
    pad_ar = jnp.arange(pad, dtype=jnp.int32)
    src2d = jnp.concatenate([src, pad_ar % n]).reshape(-1, _CHUNK)
    dst2d = jnp.concatenate(
        [dst, n + pad_ar % (n_acc - n)]).reshape(-1, _CHUNK)

    degp = _deg_count_sc(dst2d, n_acc, ngrp, h)
    deglin = degp.reshape(2, nr, 128)

    x3 = jnp.pad(x, ((0, n_acc - n), (0, 0))).reshape(nr, 8, fin)
    w_cols = [jnp.zeros((fin, 128), jnp.float32)
              .at[:, 16 * k:16 * k + h].set(W_in) for k in range(8)]
    b128 = jnp.tile(b_in, 8).reshape(1, 128)
    wo512 = _block_diag8(W_out)
    bo512 = jnp.tile(b_out, 8).reshape(1, 8 * co)

    bn8 = 1568
    hlin, glin, dinvlin = _init_tc(x3, w_cols, b128, deglin, bn8)
    h0lin = hlin
    y = None
    for i in range(n_layers):
        part = _seg_sum_sc(glin.reshape(n_acc, h), src2d, dst2d,
                           n_acc, ngrp, h)
        partlin = part.reshape(2, nr, 128)
        beta = float(np.log(_THETA / (i + 1) + 1.0))
        w128 = _block_diag8(W_layers[i])
        if i + 1 < n_layers:
            hlin, glin = _layer_tc(partlin, hlin, h0lin, dinvlin,
                                   w128, beta, bn8)
        else:
            y = _final_tc(partlin, hlin, h0lin, dinvlin, w128,
                          wo512, bo512, beta, bn8)
    return y.reshape(n_acc, co)[:n]


# speedup vs baseline: 1.2042x; 1.0223x over previous
import functools

import numpy as np
import jax
import jax.numpy as jnp
from jax import lax
from jax.experimental import pallas as pl
from jax.experimental.pallas import tpu as pltpu
from jax.experimental.pallas import tpu_sc as plsc

_ALPHA = 0.5
_THETA = 1.0

_NC = 2
_NS = 16
_NW = _NC * _NS
_CHUNK = 128
_SUB = 8


def _elu(z):
    return jnp.where(z > 0.0, z, jnp.exp(jnp.minimum(z, 0.0)) - 1.0)


def _acc_rows(n):
    per_tile = ((n + 1 + _NS * _CHUNK - 1) // (_NS * _CHUNK)) * _CHUNK
    return _NS * per_tile


def _deg_count_sc(dst2d, n_acc, ngrp):
    mesh = plsc.VectorSubcoreMesh(core_axis_name="c", subcore_axis_name="s")
    zpt = n_acc // _NS

    scratch = (
        [pltpu.VMEM((_SUB, _CHUNK), jnp.int32) for _ in range(2)]
        + [pltpu.VMEM((_CHUNK,), jnp.float32),
           pltpu.VMEM((zpt,), jnp.float32),
           pltpu.VMEM_SHARED((n_acc,), jnp.float32)]
        + [pltpu.SemaphoreType.DMA for _ in range(3)]
    )

    @functools.partial(
        pl.kernel,
        out_type=jax.ShapeDtypeStruct((_NC, n_acc), jnp.float32),
        mesh=mesh,
        scratch_types=scratch,
        compiler_params=pltpu.CompilerParams(use_tc_tiling_on_sc=False),
    )
    def deg_kernel(dst_hbm, out_hbm, dst_a, dst_b, ones_v, zflat, acc,
                   ssem, isem_a, isem_b):
        c = lax.axis_index("c")
        s = lax.axis_index("s")
        w = s * _NC + c

        zero16 = jnp.zeros((16,), jnp.float32)
        ones16 = jnp.ones((16,), jnp.float32)

        @pl.loop(0, _CHUNK // 16)
        def _(i):
            ones_v[pl.ds(i * 16, 16)] = ones16

        @pl.loop(0, zpt // 16)
        def _(i):
            zflat[pl.ds(i * 16, 16)] = zero16

        pltpu.sync_copy(zflat, acc.at[pl.ds(s * zpt, zpt)])

        plsc.subcore_barrier()

        cbase = w * ngrp * _SUB

        def fire_idx(gidx, dbuf, sem):
            pltpu.async_copy(dst_hbm.at[pl.ds(cbase + gidx * _SUB, _SUB)],
                             dbuf, sem)

        def wait_idx(dbuf, sem):
            pltpu.make_async_copy(dst_hbm.at[pl.ds(0, _SUB)], dbuf, sem).wait()

        def drain_scatters():
            for j in range(_SUB):
                pltpu.make_async_copy(
                    dst_hbm.at[pl.ds(0, _SUB)], ones_v, ssem).wait()

        def run_group(dbuf):
            for j in range(_SUB):
                pltpu.async_copy(ones_v, acc.at[dbuf.at[j]], ssem, add=True)

        fire_idx(0, dst_a, isem_a)

        @pl.loop(0, ngrp // 2)
        def _(i):
            g_a = 2 * i
            wait_idx(dst_a, isem_a)

            @pl.when(i > 0)
            def _():
                drain_scatters()

            fire_idx(g_a + 1, dst_b, isem_b)
            run_group(dst_a)
            wait_idx(dst_b, isem_b)
            drain_scatters()
            fire_idx(jnp.minimum(g_a + 2, ngrp - 1), dst_a, isem_a)
            run_group(dst_b)

        drain_scatters()
        wait_idx(dst_a, isem_a)
        plsc.subcore_barrier()
        pltpu.sync_copy(acc.at[pl.ds(s * zpt, zpt)],
                        out_hbm.at[c, pl.ds(s * zpt, zpt)])

    return deg_kernel(dst2d)


def _seg_sum_sc(g, src2d, dst2d, n_acc, ngrp, h):
    mesh = plsc.VectorSubcoreMesh(core_axis_name="c", subcore_axis_name="s")
    zpt = n_acc // _NS

    scratch = (
        [pltpu.VMEM((_SUB, _CHUNK), jnp.int32) for _ in range(4)]
        + [pltpu.VMEM((_CHUNK, h), jnp.float32) for _ in range(_SUB)]
        + [pltpu.VMEM((_CHUNK, h), jnp.float32),
           pltpu.VMEM_SHARED((n_acc, h), jnp.float32)]
        + [pltpu.SemaphoreType.DMA for _ in range(_SUB + 3)]
    )

    @functools.partial(
        pl.kernel,
        out_type=jax.ShapeDtypeStruct((_NC, n_acc, h), jnp.float32),
        mesh=mesh,
        scratch_types=scratch,
        compiler_params=pltpu.CompilerParams(use_tc_tiling_on_sc=False),
    )
    def seg_kernel(g_hbm, src_hbm, dst_hbm, out_hbm, *scr):
        src_a, dst_a, src_b, dst_b = scr[0:4]
        bufs = scr[4:4 + _SUB]
        zrows = scr[4 + _SUB]
        acc = scr[5 + _SUB]
        gsems = scr[6 + _SUB:6 + 2 * _SUB]
        ssem = scr[6 + 2 * _SUB]
        isem_a = scr[7 + 2 * _SUB]
        isem_b = scr[8 + 2 * _SUB]

        c = lax.axis_index("c")
        s = lax.axis_index("s")
        w = s * _NC + c

        zero16 = jnp.zeros((16,), jnp.float32)

        @pl.loop(0, _CHUNK)
        def _(i):
            zrows[i, :] = zero16

        @pl.loop(0, zpt // _CHUNK)
        def _(i):
            pltpu.sync_copy(zrows, acc.at[pl.ds(s * zpt + i * _CHUNK, _CHUNK)])

        plsc.subcore_barrier()

        cbase = w * ngrp * _SUB

        def fire_idx(gidx, sbuf, dbuf, sem):
            rows = pl.ds(cbase + gidx * _SUB, _SUB)
            pltpu.async_copy(src_hbm.at[rows], sbuf, sem)
            pltpu.async_copy(dst_hbm.at[rows], dbuf, sem)

        def wait_idx(sbuf, dbuf, sem):
            pltpu.make_async_copy(src_hbm.at[pl.ds(0, _SUB)], sbuf, sem).wait()
            pltpu.make_async_copy(src_hbm.at[pl.ds(0, _SUB)], dbuf, sem).wait()

        def drain_scatters():
            for j in range(_SUB):
                pltpu.make_async_copy(
                    g_hbm.at[pl.ds(0, _CHUNK)], bufs[j], ssem).wait()

        def run_group(sbuf, dbuf):
            descs = [pltpu.async_copy(g_hbm.at[sbuf.at[j]], bufs[j], gsems[j])
                     for j in range(_SUB)]
            for j in range(_SUB):
                descs[j].wait()
                pltpu.async_copy(bufs[j], acc.at[dbuf.at[j]], ssem, add=True)

        fire_idx(0, src_a, dst_a, isem_a)

        @pl.loop(0, ngrp // 2)
        def _(i):
            g_a = 2 * i
            wait_idx(src_a, dst_a, isem_a)

            @pl.when(i > 0)
            def _():
                drain_scatters()

            fire_idx(g_a + 1, src_b, dst_b, isem_b)
            run_group(src_a, dst_a)
            wait_idx(src_b, dst_b, isem_b)
            drain_scatters()
            fire_idx(jnp.minimum(g_a + 2, ngrp - 1), src_a, dst_a, isem_a)
            run_group(src_b, dst_b)

        drain_scatters()
        wait_idx(src_a, dst_a, isem_a)
        plsc.subcore_barrier()
        pltpu.sync_copy(acc.at[pl.ds(s * zpt, zpt)],
                        out_hbm.at[c, pl.ds(s * zpt, zpt)])

    return seg_kernel(g, src2d, dst2d)


def _init_tc(x3, w_cols, b128, bn8):
    nr = x3.shape[0]

    def body(*refs):
        x_b = refs[0]
        wks = refs[1:9]
        b_b, h_b = refs[9], refs[10]
        acc = jnp.broadcast_to(b_b[:], (x_b.shape[0], 128))
        for k in range(8):
            acc = acc + jnp.dot(x_b[:, k, :], wks[k][:],
                                preferred_element_type=jnp.float32)
        h_b[:] = _elu(acc)

    w_specs = [pl.BlockSpec((128, 128), lambda i: (0, 0)) for _ in range(8)]
    return pl.pallas_call(
        body,
        grid=(nr // bn8,),
        in_specs=[pl.BlockSpec((bn8, 8, 128), lambda i: (i, 0, 0))]
        + w_specs + [pl.BlockSpec((1, 128), lambda i: (0, 0))],
        out_specs=pl.BlockSpec((bn8, 128), lambda i: (i, 0)),
        out_shape=jax.ShapeDtypeStruct((nr, 128), jnp.float32),
    )(x3, *w_cols, b128)


def _prep_tc(deg2, h0wide, sbig, bnp):
    nw = deg2.shape[1]

    def body(dg_b, h0_b, s_b, di_b, g_b):
        dinv = lax.rsqrt(dg_b[0] + dg_b[1] + 1.0)
        rep = jnp.dot(dinv, s_b[:], preferred_element_type=jnp.float32)
        di_b[:] = rep
        g_b[:] = rep * h0_b[:]

    row = lambda i: (i, 0)
    return pl.pallas_call(
        body,
        grid=(nw // bnp,),
        in_specs=[
            pl.BlockSpec((2, bnp, 128), lambda i: (0, i, 0)),
            pl.BlockSpec((bnp, 2048), row),
            pl.BlockSpec((128, 2048), lambda i: (0, 0)),
        ],
        out_specs=[pl.BlockSpec((bnp, 2048), row)] * 2,
        out_shape=[jax.ShapeDtypeStruct((nw, 2048), jnp.float32)] * 2,
    )(deg2, h0wide, sbig)


def _layer_tc(partlin, hlin, h0lin, dinvlin, w128, beta, bn8):
    nr = hlin.shape[0]

    def body(p_b, h_b, h0_b, di_b, w_b, hn_b, gn_b):
        dinv = di_b[:]
        ah = dinv * (p_b[0] + p_b[1]) + (dinv * dinv) * h_b[:]
        hh = (1.0 - _ALPHA) * ah + _ALPHA * h0_b[:]
        out = (1.0 - beta) * hh + beta * jnp.dot(
            hh, w_b[:], preferred_element_type=jnp.float32)
        hn = _elu(out) + out
        hn_b[:] = hn
        gn_b[:] = dinv * hn

    row = lambda i: (i, 0)
    return pl.pallas_call(
        body,
        grid=(nr // bn8,),
        in_specs=[
            pl.BlockSpec((2, bn8, 128), lambda i: (0, i, 0)),
            pl.BlockSpec((bn8, 128), row),
            pl.BlockSpec((bn8, 128), row),
            pl.BlockSpec((bn8, 128), row),
            pl.BlockSpec((128, 128), lambda i: (0, 0)),
        ],
        out_specs=[pl.BlockSpec((bn8, 128), row), pl.BlockSpec((bn8, 128), row)],
        out_shape=[jax.ShapeDtypeStruct((nr, 128), jnp.float32)] * 2,
    )(partlin, hlin, h0lin, dinvlin, w128)


def _final_tc(partlin, hlin, h0lin, dinvlin, w128, wo512, bo512, beta, bn8):
    nr = hlin.shape[0]

    def body(p_b, h_b, h0_b, di_b, w_b, wo_b, bo_b, y_b):
        dinv = di_b[:]
        ah = dinv * (p_b[0] + p_b[1]) + (dinv * dinv) * h_b[:]
        hh = (1.0 - _ALPHA) * ah + _ALPHA * h0_b[:]
        out = (1.0 - beta) * hh + beta * jnp.dot(
            hh, w_b[:], preferred_element_type=jnp.float32)
        hn = _elu(out) + out
        y_b[:] = jnp.dot(hn, wo_b[:],
                         preferred_element_type=jnp.float32) + bo_b[:]

    row = lambda i: (i, 0)
    return pl.pallas_call(
        body,
        grid=(nr // bn8,),
        in_specs=[
            pl.BlockSpec((2, bn8, 128), lambda i: (0, i, 0)),
            pl.BlockSpec((bn8, 128), row),
            pl.BlockSpec((bn8, 128), row),
            pl.BlockSpec((bn8, 128), row),
            pl.BlockSpec((128, 128), lambda i: (0, 0)),
            pl.BlockSpec((128, 512), lambda i: (0, 0)),
            pl.BlockSpec((1, 512), lambda i: (0, 0)),
        ],
        out_specs=pl.BlockSpec((bn8, 512), row),
        out_shape=jax.ShapeDtypeStruct((nr, 512), jnp.float32),
    )(partlin, hlin, h0lin, dinvlin, w128, wo512, bo512)


def _block_diag8(w):
    a, b = w.shape
    out = jnp.zeros((8 * a, 8 * b), w.dtype)
    for k in range(8):
        out = out.at[k * a:(k + 1) * a, k * b:(k + 1) * b].set(w)
    return out


def kernel(x, edge_index, W_in, b_in, W_layers, W_out, b_out):
    n, fin = x.shape
    e = edge_index.shape[1]
    h = W_in.shape[1]
    n_layers = W_layers.shape[0]
    co = W_out.shape[1]

    grand = _NW * _SUB * _CHUNK
    ngrp = (e + grand - 1) // grand
    ngrp += ngrp % 2
    e_pad = ngrp * grand
    n_acc = _acc_rows(n)
    nr = n_acc * h // 128

    src = edge_index[0]
    dst = edge_index[1]
    pad = e_pad - e
    pad_ar = jnp.arange(pad, dtype=jnp.int32)
    src2d = jnp.concatenate([src, pad_ar % n]).reshape(-1, _CHUNK)
    dst2d = jnp.concatenate(
        [dst, n + pad_ar % (n_acc - n)]).reshape(-1, _CHUNK)

    degp = _deg_count_sc(dst2d, n_acc, ngrp)
    deg2 = degp.reshape(2, n_acc // 128, 128)

    x3 = jnp.pad(x, ((0, n_acc - n), (0, 0))).reshape(nr, 8, fin)
    w_cols = [jnp.zeros((fin, 128), jnp.float32)
              .at[:, 16 * k:16 * k + h].set(W_in) for k in range(8)]
    b128 = jnp.tile(b_in, 8).reshape(1, 128)
    wo512 = _block_diag8(W_out)
    bo512 = jnp.tile(b_out, 8).reshape(1, 8 * co)
    sb = np.zeros((128, 2048), np.float32)
    for j in range(2048):
        sb[8 * (j // 128) + (j % 128) // 16, j] = 1.0
    sbig = jnp.asarray(sb)

    bn8 = 1568
    hlin = _init_tc(x3, w_cols, b128, bn8)
    h0lin = hlin
    dinvwide, g0wide = _prep_tc(deg2, hlin.reshape(n_acc // 128, 2048),
                                sbig, n_acc // 128)
    dinvlin = dinvwide.reshape(nr, 128)
    glin = g0wide.reshape(nr, 128)
    y = None
    for i in range(n_layers):
        part = _seg_sum_sc(glin.reshape(n_acc, h), src2d, dst2d,
                           n_acc, ngrp, h)
        partlin = part.reshape(2, nr, 128)
        beta = float(np.log(_THETA / (i + 1) + 1.0))
        w128 = _block_diag8(W_layers[i])
        if i + 1 < n_layers:
            hlin, glin = _layer_tc(partlin, hlin, h0lin, dinvlin,
                                   w128, beta, bn8)
        else:
            y = _final_tc(partlin, hlin, h0lin, dinvlin, w128,
                          wo512, bo512, beta, bn8)
    return y.reshape(n_acc, co)[:n]


# speedup vs baseline: 1.2244x; 1.0168x over previous
import functools

import numpy as np
import jax
import jax.numpy as jnp
from jax import lax
from jax.experimental import pallas as pl
from jax.experimental.pallas import tpu as pltpu
from jax.experimental.pallas import tpu_sc as plsc

_ALPHA = 0.5
_THETA = 1.0

_NC = 2
_NS = 16
_NW = _NC * _NS
_CHUNK = 128
_SUB = 8


def _elu(z):
    return jnp.where(z > 0.0, z, jnp.exp(jnp.minimum(z, 0.0)) - 1.0)


def _acc_rows(n):
    per_tile = ((n + 1 + _NS * _CHUNK - 1) // (_NS * _CHUNK)) * _CHUNK
    return _NS * per_tile


def _deg_count_sc(dst2d, n_acc, ngrp):
    mesh = plsc.VectorSubcoreMesh(core_axis_name="c", subcore_axis_name="s")
    zpt = n_acc // _NS

    scratch = (
        [pltpu.VMEM((_SUB, _CHUNK), jnp.int32) for _ in range(2)]
        + [pltpu.VMEM((_CHUNK,), jnp.float32),
           pltpu.VMEM((zpt,), jnp.float32),
           pltpu.VMEM_SHARED((n_acc,), jnp.float32)]
        + [pltpu.SemaphoreType.DMA for _ in range(3)]
    )

    @functools.partial(
        pl.kernel,
        out_type=jax.ShapeDtypeStruct((_NC, n_acc), jnp.float32),
        mesh=mesh,
        scratch_types=scratch,
        compiler_params=pltpu.CompilerParams(use_tc_tiling_on_sc=False),
    )
    def deg_kernel(dst_hbm, out_hbm, dst_a, dst_b, ones_v, zflat, acc,
                   ssem, isem_a, isem_b):
        c = lax.axis_index("c")
        s = lax.axis_index("s")
        w = s * _NC + c

        zero16 = jnp.zeros((16,), jnp.float32)
        ones16 = jnp.ones((16,), jnp.float32)

        @pl.loop(0, _CHUNK // 16)
        def _(i):
            ones_v[pl.ds(i * 16, 16)] = ones16

        @pl.loop(0, zpt // 16)
        def _(i):
            zflat[pl.ds(i * 16, 16)] = zero16

        pltpu.sync_copy(zflat, acc.at[pl.ds(s * zpt, zpt)])

        plsc.subcore_barrier()

        cbase = w * ngrp * _SUB

        def fire_idx(gidx, dbuf, sem):
            pltpu.async_copy(dst_hbm.at[pl.ds(cbase + gidx * _SUB, _SUB)],
                             dbuf, sem)

        def wait_idx(dbuf, sem):
            pltpu.make_async_copy(dst_hbm.at[pl.ds(0, _SUB)], dbuf, sem).wait()

        def drain_scatters():
            for j in range(_SUB):
                pltpu.make_async_copy(
                    dst_hbm.at[pl.ds(0, _SUB)], ones_v, ssem).wait()

        def run_group(dbuf):
            for j in range(_SUB):
                pltpu.async_copy(ones_v, acc.at[dbuf.at[j]], ssem, add=True)

        fire_idx(0, dst_a, isem_a)

        @pl.loop(0, ngrp // 2)
        def _(i):
            g_a = 2 * i
            wait_idx(dst_a, isem_a)

            @pl.when(i > 0)
            def _():
                drain_scatters()

            fire_idx(g_a + 1, dst_b, isem_b)
            run_group(dst_a)
            wait_idx(dst_b, isem_b)
            drain_scatters()
            fire_idx(jnp.minimum(g_a + 2, ngrp - 1), dst_a, isem_a)
            run_group(dst_b)

        drain_scatters()
        wait_idx(dst_a, isem_a)
        plsc.subcore_barrier()
        pltpu.sync_copy(acc.at[pl.ds(s * zpt, zpt)],
                        out_hbm.at[c, pl.ds(s * zpt, zpt)])

    return deg_kernel(dst2d)


def _seg_sum_sc(g, src2d, dst2d, n_acc, ngrp, h):
    mesh = plsc.VectorSubcoreMesh(core_axis_name="c", subcore_axis_name="s")
    zpt = n_acc // _NS

    scratch = (
        [pltpu.VMEM((_SUB, _CHUNK), jnp.int32) for _ in range(4)]
        + [pltpu.VMEM((_CHUNK, h), jnp.float32) for _ in range(_SUB)]
        + [pltpu.VMEM((_CHUNK, h), jnp.float32),
           pltpu.VMEM_SHARED((n_acc, h), jnp.float32)]
        + [pltpu.SemaphoreType.DMA for _ in range(_SUB + 3)]
    )

    @functools.partial(
        pl.kernel,
        out_type=jax.ShapeDtypeStruct((_NC, n_acc, h), jnp.float32),
        mesh=mesh,
        scratch_types=scratch,
        compiler_params=pltpu.CompilerParams(use_tc_tiling_on_sc=False),
    )
    def seg_kernel(g_hbm, src_hbm, dst_hbm, out_hbm, *scr):
        src_a, dst_a, src_b, dst_b = scr[0:4]
        bufs = scr[4:4 + _SUB]
        zrows = scr[4 + _SUB]
        acc = scr[5 + _SUB]
        gsems = scr[6 + _SUB:6 + 2 * _SUB]
        ssem = scr[6 + 2 * _SUB]
        isem_a = scr[7 + 2 * _SUB]
        isem_b = scr[8 + 2 * _SUB]

        c = lax.axis_index("c")
        s = lax.axis_index("s")
        w = s * _NC + c

        zero16 = jnp.zeros((16,), jnp.float32)

        @pl.loop(0, _CHUNK)
        def _(i):
            zrows[i, :] = zero16

        @pl.loop(0, zpt // _CHUNK)
        def _(i):
            pltpu.sync_copy(zrows, acc.at[pl.ds(s * zpt + i * _CHUNK, _CHUNK)])

        plsc.subcore_barrier()

        cbase = w * ngrp * _SUB

        def fire_idx(gidx, sbuf, dbuf, sem):
            rows = pl.ds(cbase + gidx * _SUB, _SUB)
            pltpu.async_copy(src_hbm.at[rows], sbuf, sem)
            pltpu.async_copy(dst_hbm.at[rows], dbuf, sem)

        def wait_idx(sbuf, dbuf, sem):
            pltpu.make_async_copy(src_hbm.at[pl.ds(0, _SUB)], sbuf, sem).wait()
            pltpu.make_async_copy(src_hbm.at[pl.ds(0, _SUB)], dbuf, sem).wait()

        def drain_scatters():
            for j in range(_SUB):
                pltpu.make_async_copy(
                    g_hbm.at[pl.ds(0, _CHUNK)], bufs[j], ssem).wait()

        def run_group(sbuf, dbuf):
            descs = [pltpu.async_copy(g_hbm.at[sbuf.at[j]], bufs[j], gsems[j])
                     for j in range(_SUB)]
            for j in range(_SUB):
                descs[j].wait()
                pltpu.async_copy(bufs[j], acc.at[dbuf.at[j]], ssem, add=True)

        fire_idx(0, src_a, dst_a, isem_a)

        @pl.loop(0, ngrp // 2)
        def _(i):
            g_a = 2 * i
            wait_idx(src_a, dst_a, isem_a)

            @pl.when(i > 0)
            def _():
                drain_scatters()

            fire_idx(g_a + 1, src_b, dst_b, isem_b)
            run_group(src_a, dst_a)
            wait_idx(src_b, dst_b, isem_b)
            drain_scatters()
            fire_idx(jnp.minimum(g_a + 2, ngrp - 1), src_a, dst_a, isem_a)
            run_group(src_b, dst_b)

        drain_scatters()
        wait_idx(src_a, dst_a, isem_a)
        plsc.subcore_barrier()
        pltpu.sync_copy(acc.at[pl.ds(s * zpt, zpt)],
                        out_hbm.at[c, pl.ds(s * zpt, zpt)])

    return seg_kernel(g, src2d, dst2d)


def _init_tc(x3, w_cols, b128, bn8, nr):
    def body(*refs):
        x_b = refs[0]
        wks = refs[1:9]
        b_b, h_b = refs[9], refs[10]
        acc = jnp.broadcast_to(b_b[:], (x_b.shape[0], 128))
        for k in range(8):
            acc = acc + jnp.dot(x_b[:, k, :], wks[k][:],
                                preferred_element_type=jnp.float32)
        h_b[:] = _elu(acc)

    w_specs = [pl.BlockSpec((128, 128), lambda i: (0, 0)) for _ in range(8)]
    return pl.pallas_call(
        body,
        grid=(nr // bn8,),
        in_specs=[pl.BlockSpec((bn8, 8, 128), lambda i: (i, 0, 0))]
        + w_specs + [pl.BlockSpec((1, 128), lambda i: (0, 0))],
        out_specs=pl.BlockSpec((bn8, 128), lambda i: (i, 0)),
        out_shape=jax.ShapeDtypeStruct((nr, 128), jnp.float32),
    )(x3, *w_cols, b128)


def _prep_tc(deg2, h0wide, sbig, bnp):
    nw = deg2.shape[1]

    def body(dg_b, h0_b, s_b, di_b, g_b):
        dinv = lax.rsqrt(dg_b[0] + dg_b[1] + 1.0)
        rep = jnp.dot(dinv, s_b[:], preferred_element_type=jnp.float32)
        di_b[:] = rep
        g_b[:] = rep * h0_b[:]

    row = lambda i: (i, 0)
    return pl.pallas_call(
        body,
        grid=(nw // bnp,),
        in_specs=[
            pl.BlockSpec((2, bnp, 128), lambda i: (0, i, 0)),
            pl.BlockSpec((bnp, 2048), row),
            pl.BlockSpec((128, 2048), lambda i: (0, 0)),
        ],
        out_specs=[pl.BlockSpec((bnp, 2048), row)] * 2,
        out_shape=[jax.ShapeDtypeStruct((nw, 2048), jnp.float32)] * 2,
    )(deg2, h0wide, sbig)


def _layer_tc(partlin, hlin, h0lin, dinvlin, w128, beta, bn8):
    nr = hlin.shape[0]

    def body(p_b, h_b, h0_b, di_b, w_b, hn_b, gn_b):
        dinv = di_b[:]
        ah = dinv * (p_b[0] + p_b[1]) + (dinv * dinv) * h_b[:]
        hh = (1.0 - _ALPHA) * ah + _ALPHA * h0_b[:]
        out = (1.0 - beta) * hh + beta * jnp.dot(
            hh, w_b[:], preferred_element_type=jnp.float32)
        hn = _elu(out) + out
        hn_b[:] = hn
        gn_b[:] = dinv * hn

    row = lambda i: (i, 0)
    return pl.pallas_call(
        body,
        grid=(nr // bn8,),
        in_specs=[
            pl.BlockSpec((2, bn8, 128), lambda i: (0, i, 0)),
            pl.BlockSpec((bn8, 128), row),
            pl.BlockSpec((bn8, 128), row),
            pl.BlockSpec((bn8, 128), row),
            pl.BlockSpec((128, 128), lambda i: (0, 0)),
        ],
        out_specs=[pl.BlockSpec((bn8, 128), row), pl.BlockSpec((bn8, 128), row)],
        out_shape=[jax.ShapeDtypeStruct((nr, 128), jnp.float32)] * 2,
    )(partlin, hlin, h0lin, dinvlin, w128)


def _final_tc(partlin, hlin, h0lin, dinvlin, w128, wo512, bo512, beta, bn8):
    nr = hlin.shape[0]

    def body(p_b, h_b, h0_b, di_b, w_b, wo_b, bo_b, y_b):
        dinv = di_b[:]
        ah = dinv * (p_b[0] + p_b[1]) + (dinv * dinv) * h_b[:]
        hh = (1.0 - _ALPHA) * ah + _ALPHA * h0_b[:]
        out = (1.0 - beta) * hh + beta * jnp.dot(
            hh, w_b[:], preferred_element_type=jnp.float32)
        hn = _elu(out) + out
        y_b[:] = jnp.dot(hn, wo_b[:],
                         preferred_element_type=jnp.float32) + bo_b[:]

    row = lambda i: (i, 0)
    return pl.pallas_call(
        body,
        grid=(nr // bn8,),
        in_specs=[
            pl.BlockSpec((2, bn8, 128), lambda i: (0, i, 0)),
            pl.BlockSpec((bn8, 128), row),
            pl.BlockSpec((bn8, 128), row),
            pl.BlockSpec((bn8, 128), row),
            pl.BlockSpec((128, 128), lambda i: (0, 0)),
            pl.BlockSpec((128, 512), lambda i: (0, 0)),
            pl.BlockSpec((1, 512), lambda i: (0, 0)),
        ],
        out_specs=pl.BlockSpec((bn8, 512), row),
        out_shape=jax.ShapeDtypeStruct((nr, 512), jnp.float32),
    )(partlin, hlin, h0lin, dinvlin, w128, wo512, bo512)


def _block_diag8(w):
    a, b = w.shape
    out = jnp.zeros((8 * a, 8 * b), w.dtype)
    for k in range(8):
        out = out.at[k * a:(k + 1) * a, k * b:(k + 1) * b].set(w)
    return out


def kernel(x, edge_index, W_in, b_in, W_layers, W_out, b_out):
    n, fin = x.shape
    e = edge_index.shape[1]
    h = W_in.shape[1]
    n_layers = W_layers.shape[0]
    co = W_out.shape[1]

    grand = _NW * _SUB * _CHUNK
    ngrp = (e + grand - 1) // grand
    ngrp += ngrp % 2
    e_pad = ngrp * grand
    n_acc = _acc_rows(n)
    nr = n_acc * h // 128

    src = edge_index[0]
    dst = edge_index[1]
    pad = e_pad - e
    pad_ar = jnp.arange(pad, dtype=jnp.int32)
    src2d = jnp.concatenate([src, pad_ar % n]).reshape(-1, _CHUNK)
    dst2d = jnp.concatenate(
        [dst, n + pad_ar % (n_acc - n)]).reshape(-1, _CHUNK)

    degp = _deg_count_sc(dst2d, n_acc, ngrp)
    deg2 = degp.reshape(2, n_acc // 128, 128)

    x3 = x.reshape(n // 8, 8, fin)
    w_cols = [jnp.zeros((fin, 128), jnp.float32)
              .at[:, 16 * k:16 * k + h].set(W_in) for k in range(8)]
    b128 = jnp.tile(b_in, 8).reshape(1, 128)
    wo512 = _block_diag8(W_out)
    bo512 = jnp.tile(b_out, 8).reshape(1, 8 * co)
    sb = np.zeros((128, 2048), np.float32)
    for j in range(2048):
        sb[8 * (j // 128) + (j % 128) // 16, j] = 1.0
    sbig = jnp.asarray(sb)

    bn8 = 1568
    hlin = _init_tc(x3, w_cols, b128, bn8, nr)
    h0lin = hlin
    dinvwide, g0wide = _prep_tc(deg2, hlin.reshape(n_acc // 128, 2048),
                                sbig, n_acc // 128)
    dinvlin = dinvwide.reshape(nr, 128)
    glin = g0wide.reshape(nr, 128)
    y = None
    for i in range(n_layers):
        part = _seg_sum_sc(glin.reshape(n_acc, h), src2d, dst2d,
                           n_acc, ngrp, h)
        partlin = part.reshape(2, nr, 128)
        beta = float(np.log(_THETA / (i + 1) + 1.0))
        w128 = _block_diag8(W_layers[i])
        if i + 1 < n_layers:
            hlin, glin = _layer_tc(partlin, hlin, h0lin, dinvlin,
                                   w128, beta, bn8)
        else:
            y = _final_tc(partlin, hlin, h0lin, dinvlin, w128,
                          wo512, bo512, beta, bn8)
    return y.reshape(n_acc, co)[:n]
